# baseline (device time: 256052 ns/iter reference)
import jax
import jax.numpy as jnp
from jax import lax
from jax.experimental import pallas as pl
from jax.experimental.pallas import tpu as pltpu

N_DEV = 16
B = 2
SQ = 256
SKV = 256
HQ = 4
DH = 64
DMODEL = 512
DQK = HQ * DH
BLK = 64


def kernel(x, Wq, K_ext, V_ext, Wo):
    def body(x_ref, wq_ref, k_ref, v_ref, wo_ref, out_ref,
             kv_all, send_sems, recv_sems):
        my_pos = lax.axis_index("i")
        left = lax.rem(my_pos - 1 + N_DEV, N_DEV)
        right = lax.rem(my_pos + 1, N_DEV)

        barrier_sem = pltpu.get_barrier_semaphore()
        for nbr in (left, right):
            pl.semaphore_signal(
                barrier_sem, inc=1,
                device_id=(nbr,), device_id_type=pl.DeviceIdType.MESH,
            )
        pl.semaphore_wait(barrier_sem, 2)

        kv_all[my_pos, 0] = k_ref[...].astype(jnp.bfloat16)
        kv_all[my_pos, 1] = v_ref[...].astype(jnp.bfloat16)

        for h in range(N_DEV - 1):
            src_slot = lax.rem(my_pos - h + N_DEV, N_DEV)
            dst_slot = src_slot
            rdma = pltpu.make_async_remote_copy(
                src_ref=kv_all.at[src_slot],
                dst_ref=kv_all.at[dst_slot],
                send_sem=send_sems.at[h],
                recv_sem=recv_sems.at[h],
                device_id=(right,),
                device_id_type=pl.DeviceIdType.MESH,
            )
            rdma.start()
            rdma.wait()

        rows = lax.broadcasted_iota(jnp.int32, (SQ, N_DEV * SKV), 0) \
            + my_pos * SQ
        cols = lax.broadcasted_iota(jnp.int32, (SQ, N_DEV * SKV), 1)
        mask = (cols // BLK) <= (rows // BLK)

        wq = wq_ref[...].astype(jnp.bfloat16)
        for b in range(B):
            q_b = jnp.dot(x_ref[b].astype(jnp.bfloat16), wq,
                          preferred_element_type=jnp.float32)
            q_b = q_b.astype(jnp.bfloat16)
            acc = jnp.zeros((SQ, DMODEL), jnp.float32)
            for h in range(HQ):
                q_bh = q_b[:, h * DH:(h + 1) * DH]
                k_bh = kv_all[:, 0, b, :, h, :].reshape(N_DEV * SKV, DH)
                v_bh = kv_all[:, 1, b, :, h, :].reshape(N_DEV * SKV, DH)
                scores = jax.lax.dot_general(
                    q_bh, k_bh,
                    dimension_numbers=(((1,), (1,)), ((), ())),
                    preferred_element_type=jnp.float32,
                ) * 0.125
                scores = jnp.where(mask, scores, -1e9)
                m = jnp.max(scores, axis=-1, keepdims=True)
                w = jnp.exp(scores - m)
                w = w / jnp.sum(w, axis=-1, keepdims=True)
                ctx = jnp.dot(w.astype(jnp.bfloat16), v_bh,
                              preferred_element_type=jnp.float32)
                acc = acc + jnp.dot(
                    ctx.astype(jnp.bfloat16),
                    wo_ref[h * DH:(h + 1) * DH, :].astype(jnp.bfloat16),
                    preferred_element_type=jnp.float32)
            out_ref[b] = acc

    return pl.pallas_call(
        body,
        out_shape=jax.ShapeDtypeStruct((B, SQ, DMODEL), jnp.float32),
        in_specs=[pl.BlockSpec(memory_space=pltpu.VMEM)] * 5,
        out_specs=pl.BlockSpec(memory_space=pltpu.VMEM),
        scratch_shapes=[
            pltpu.VMEM((N_DEV, 2, B, SKV, HQ, DH), jnp.bfloat16),
            pltpu.SemaphoreType.DMA((N_DEV - 1,)),
            pltpu.SemaphoreType.DMA((N_DEV - 1,)),
        ],
        compiler_params=pltpu.CompilerParams(collective_id=0),
    )(x, Wq, K_ext, V_ext, Wo)


# device time: 205945 ns/iter; 1.2433x vs baseline; 1.2433x over previous
import jax
import jax.numpy as jnp
from jax import lax
from jax.experimental import pallas as pl
from jax.experimental.pallas import tpu as pltpu

N_DEV = 16
B = 2
SQ = 256
SKV = 256
HQ = 4
DH = 64
DMODEL = 512
BLK = 64


def kernel(x, Wq, K_ext, V_ext, Wo):
    def body(x_ref, wq_ref, k_ref, v_ref, wo_ref, out_ref,
             kv_all, send_sems, recv_sems):
        my_pos = lax.axis_index("i")
        left = lax.rem(my_pos - 1 + N_DEV, N_DEV)
        right = lax.rem(my_pos + 1, N_DEV)

        barrier_sem = pltpu.get_barrier_semaphore()
        for nbr in (left, right):
            pl.semaphore_signal(
                barrier_sem, inc=1,
                device_id=(nbr,), device_id_type=pl.DeviceIdType.MESH,
            )
        pl.semaphore_wait(barrier_sem, 2)

        kv_all[my_pos, 0] = jnp.transpose(
            k_ref[...].astype(jnp.bfloat16), (0, 2, 1, 3)).reshape(
                B * HQ, SKV, DH)
        kv_all[my_pos, 1] = jnp.transpose(
            v_ref[...].astype(jnp.bfloat16), (0, 2, 1, 3)).reshape(
                B * HQ, SKV, DH)

        wq = wq_ref[...].astype(jnp.bfloat16)
        q_all = jnp.stack([
            jnp.transpose(
                jnp.dot(x_ref[b].astype(jnp.bfloat16), wq,
                        preferred_element_type=jnp.float32)
                .astype(jnp.bfloat16).reshape(SQ, HQ, DH),
                (1, 0, 2))
            for b in range(B)
        ]).reshape(B * HQ, SQ, DH)

        li = lax.broadcasted_iota(jnp.int32, (SQ, SKV), 0)
        lj = lax.broadcasted_iota(jnp.int32, (SQ, SKV), 1)
        diag_bias = jnp.where((lj // BLK) <= (li // BLK),
                              0.0, -1e9).astype(jnp.float32)

        num = jnp.zeros((B * HQ, SQ, DH), jnp.float32)
        den = jnp.zeros((B * HQ, SQ, 1), jnp.float32)

        for s in range(N_DEV):
            slot = lax.rem(my_pos - s + N_DEV, N_DEV)
            if s < N_DEV - 1:
                rdma = pltpu.make_async_remote_copy(
                    src_ref=kv_all.at[slot],
                    dst_ref=kv_all.at[slot],
                    send_sem=send_sems.at[s],
                    recv_sem=recv_sems.at[s],
                    device_id=(right,),
                    device_id_type=pl.DeviceIdType.MESH,
                )
                rdma.start()

            k_c = kv_all[slot, 0]
            v_c = kv_all[slot, 1]
            scores = lax.dot_general(
                q_all, k_c,
                dimension_numbers=(((2,), (2,)), ((0,), (0,))),
                preferred_element_type=jnp.float32,
            ) * 0.125
            bias = jnp.where(
                slot < my_pos,
                jnp.zeros((SQ, SKV), jnp.float32),
                jnp.where(slot == my_pos, diag_bias,
                          jnp.full((SQ, SKV), -1e9, jnp.float32)),
            )
            w = jnp.exp(scores + bias[None])
            num = num + lax.dot_general(
                w.astype(jnp.bfloat16), v_c,
                dimension_numbers=(((2,), (1,)), ((0,), (0,))),
                preferred_element_type=jnp.float32,
            )
            den = den + jnp.sum(w, axis=-1, keepdims=True)

            if s < N_DEV - 1:
                rdma.wait_recv()
                rdma.wait_send()

        ctx = (num / den).reshape(B, HQ, SQ, DH)
        wo = wo_ref[...].astype(jnp.bfloat16)
        for b in range(B):
            ctx_b = jnp.transpose(ctx[b], (1, 0, 2)).reshape(SQ, HQ * DH)
            out_ref[b] = jnp.dot(ctx_b.astype(jnp.bfloat16), wo,
                                 preferred_element_type=jnp.float32)

    return pl.pallas_call(
        body,
        out_shape=jax.ShapeDtypeStruct((B, SQ, DMODEL), jnp.float32),
        in_specs=[pl.BlockSpec(memory_space=pltpu.VMEM)] * 5,
        out_specs=pl.BlockSpec(memory_space=pltpu.VMEM),
        scratch_shapes=[
            pltpu.VMEM((N_DEV, 2, B * HQ, SKV, DH), jnp.bfloat16),
            pltpu.SemaphoreType.DMA((N_DEV - 1,)),
            pltpu.SemaphoreType.DMA((N_DEV - 1,)),
        ],
        compiler_params=pltpu.CompilerParams(collective_id=0),
    )(x, Wq, K_ext, V_ext, Wo)
